# Initial kernel scaffold; baseline (speedup 1.0000x reference)
import jax
import jax.numpy as jnp
from jax.experimental import pallas as pl


def kernel(x, sym_table, pos_table):
    raise NotImplementedError("write your pallas kernel here")


# speedup vs baseline: 2.9530x; 2.9530x over previous
import functools

import jax
import jax.numpy as jnp
from jax import lax
from jax.experimental import pallas as pl
from jax.experimental.pallas import tpu as pltpu
from jax.experimental.pallas import tpu_sc as plsc

B = 1024
L = 200
D = 128
NW = 32
SEQ_PER_W = B // NW
L_PER_RND = 8
N_ROUNDS = L // L_PER_RND
ROWS_PER_XFER = 128
XFERS_PER_RND = 2
L_PER_XFER = 4

_mesh = plsc.VectorSubcoreMesh(core_axis_name="c", subcore_axis_name="s")


@functools.partial(
    pl.kernel,
    mesh=_mesh,
    out_type=jax.ShapeDtypeStruct((B * L, D), jnp.float32),
    scratch_types=[
        pltpu.VMEM((XFERS_PER_RND, ROWS_PER_XFER), jnp.int32),
        pltpu.VMEM((XFERS_PER_RND, ROWS_PER_XFER), jnp.int32),
        pltpu.VMEM((L_PER_RND, D), jnp.float32),
        pltpu.VMEM((XFERS_PER_RND, ROWS_PER_XFER, D), jnp.float32),
        pltpu.SemaphoreType.DMA,
    ],
)
def _emb_sc(xt_hbm, sym_hbm, pos_hbm, oidx_hbm, out_hbm,
            idx_v, oidx_v, pos_v, buf_v, sem):
    w = lax.axis_index("s") * 2 + lax.axis_index("c")
    rows_per_worker = N_ROUNDS * XFERS_PER_RND

    def round_body(g, carry):
        row0 = w * rows_per_worker + g * XFERS_PER_RND
        pltpu.sync_copy(xt_hbm.at[pl.ds(row0, XFERS_PER_RND)], idx_v)
        pltpu.sync_copy(oidx_hbm.at[pl.ds(row0, XFERS_PER_RND)], oidx_v)
        pltpu.sync_copy(pos_hbm.at[pl.ds(g * L_PER_RND, L_PER_RND)], pos_v)
        for t in range(XFERS_PER_RND):
            pltpu.async_copy(sym_hbm.at[idx_v.at[t]], buf_v.at[t], sem).wait()
        for t in range(XFERS_PER_RND):
            for li in range(L_PER_XFER):
                pvecs = [pos_v[t * L_PER_XFER + li, pl.ds(j * 16, 16)]
                         for j in range(8)]
                for s in range(SEQ_PER_W):
                    r = li * SEQ_PER_W + s
                    for j in range(8):
                        plsc.addupdate(buf_v.at[t, r, pl.ds(j * 16, 16)],
                                       pvecs[j])
        for t in range(XFERS_PER_RND):
            pltpu.async_copy(buf_v.at[t], out_hbm.at[oidx_v.at[t]], sem).wait()
        return carry

    lax.fori_loop(0, N_ROUNDS, round_body, 0)


def _lmajor(a):
    return a.reshape(NW, SEQ_PER_W, L).swapaxes(1, 2).reshape(
        NW * N_ROUNDS * XFERS_PER_RND, ROWS_PER_XFER)


def kernel(x, sym_table, pos_table):
    xt = _lmajor(x.astype(jnp.int32))
    oidx = _lmajor(jnp.arange(B * L, dtype=jnp.int32))
    out = _emb_sc(xt, sym_table, pos_table, oidx)
    return out.reshape(B, L, D)


# speedup vs baseline: 6.2886x; 2.1295x over previous
import functools

import jax
import jax.numpy as jnp
from jax import lax
from jax.experimental import pallas as pl
from jax.experimental.pallas import tpu as pltpu
from jax.experimental.pallas import tpu_sc as plsc

B = 1024
L = 200
D = 128
NW = 32
SEQ_PER_W = B // NW
L_PER_RND = 8
N_ROUNDS = L // L_PER_RND
ROWS_PER_XFER = 128
XFERS_PER_RND = 2
L_PER_XFER = 4
NSLOT = 3
ROWS_PER_W = N_ROUNDS * XFERS_PER_RND

_mesh = plsc.VectorSubcoreMesh(core_axis_name="c", subcore_axis_name="s")


@functools.partial(
    pl.kernel,
    mesh=_mesh,
    out_type=jax.ShapeDtypeStruct((B * L, D), jnp.float32),
    scratch_types=[
        pltpu.VMEM((NSLOT, XFERS_PER_RND, ROWS_PER_XFER), jnp.int32),
        pltpu.VMEM((NSLOT, XFERS_PER_RND, ROWS_PER_XFER), jnp.int32),
        pltpu.VMEM((NSLOT, L_PER_RND, D), jnp.float32),
        pltpu.VMEM((NSLOT, XFERS_PER_RND, ROWS_PER_XFER, D), jnp.float32),
        pltpu.SemaphoreType.DMA((NSLOT,)),
        pltpu.SemaphoreType.DMA((NSLOT,)),
        pltpu.SemaphoreType.DMA((NSLOT,)),
        pltpu.SemaphoreType.DMA((NSLOT,)),
    ],
)
def _emb_sc(xt_hbm, sym_hbm, pos_hbm, oidx_hbm, out_hbm,
            idx_v, oidx_v, pos_v, buf_v, sem_ip, sem_o, sem_g, sem_s):
    w = lax.axis_index("s") * 2 + lax.axis_index("c")

    def ip_copies(g, sl):
        row0 = w * ROWS_PER_W + g * XFERS_PER_RND
        return (
            pltpu.make_async_copy(xt_hbm.at[pl.ds(row0, XFERS_PER_RND)],
                                  idx_v.at[sl], sem_ip.at[sl]),
            pltpu.make_async_copy(pos_hbm.at[pl.ds(g * L_PER_RND, L_PER_RND)],
                                  pos_v.at[sl], sem_ip.at[sl]),
        )

    def o_copy(g, sl):
        row0 = w * ROWS_PER_W + g * XFERS_PER_RND
        return pltpu.make_async_copy(oidx_hbm.at[pl.ds(row0, XFERS_PER_RND)],
                                     oidx_v.at[sl], sem_o.at[sl])

    def g_copies(sl):
        return tuple(
            pltpu.make_async_copy(sym_hbm.at[idx_v.at[sl, t]],
                                  buf_v.at[sl, t], sem_g.at[sl])
            for t in range(XFERS_PER_RND))

    def s_copies(sl):
        return tuple(
            pltpu.make_async_copy(buf_v.at[sl, t],
                                  out_hbm.at[oidx_v.at[sl, t]], sem_s.at[sl])
            for t in range(XFERS_PER_RND))

    def start(cs):
        for c in cs:
            c.start()

    def wait(cs):
        for c in cs:
            c.wait()

    def adds(p):
        for t in range(XFERS_PER_RND):
            pvecs = [[pos_v[p, t * L_PER_XFER + li, pl.ds(j * 16, 16)]
                      for j in range(8)] for li in range(L_PER_XFER)]

            def sb_body(sb, carry, t=t, pvecs=pvecs):
                for u in range(16):
                    for li in range(L_PER_XFER):
                        base = (sb * 16 + u) * L_PER_XFER + li
                        for j in range(8):
                            plsc.addupdate(
                                buf_v.at[p, t, base, pl.ds(j * 16, 16)],
                                pvecs[li][j])
                return carry

            lax.fori_loop(0, 2, sb_body, 0)

    def round_step(g, phase):
        q = (phase + 1) % NSLOT
        wait(g_copies(phase))
        @pl.when(g + 2 < N_ROUNDS)
        def _():
            start(ip_copies(g + 2, (phase + 2) % NSLOT))
        @pl.when(g >= 2)
        def _():
            wait(s_copies(q))
        start((o_copy(g + 1, q),))
        wait(ip_copies(g + 1, q))
        start(g_copies(q))
        adds(phase)
        wait((o_copy(g, phase),))
        start(s_copies(phase))

    start(ip_copies(0, 0))
    start((o_copy(0, 0),))
    wait(ip_copies(0, 0))
    start(g_copies(0))
    start(ip_copies(1, 1))

    def body(k, carry):
        for c in range(NSLOT):
            round_step(NSLOT * k + c, c)
        return carry

    lax.fori_loop(0, (N_ROUNDS - 1) // NSLOT, body, 0)

    g_last = N_ROUNDS - 1
    wait(g_copies(0))
    adds(0)
    wait((o_copy(g_last, 0),))
    start(s_copies(0))
    wait(s_copies(1))
    wait(s_copies(2))
    wait(s_copies(0))


def _bmajor(a):
    return a.reshape(NW, SEQ_PER_W, ROWS_PER_W, L_PER_XFER).transpose(
        0, 2, 1, 3).reshape(NW * ROWS_PER_W, ROWS_PER_XFER)


def kernel(x, sym_table, pos_table):
    xt = _bmajor(x.astype(jnp.int32))
    oidx = _bmajor(jnp.arange(B * L, dtype=jnp.int32))
    out = _emb_sc(xt, sym_table, pos_table, oidx)
    return out.reshape(B, L, D)


# speedup vs baseline: 6.3326x; 1.0070x over previous
import functools

import jax
import jax.numpy as jnp
from jax import lax
from jax.experimental import pallas as pl
from jax.experimental.pallas import tpu as pltpu
from jax.experimental.pallas import tpu_sc as plsc

B = 1024
L = 200
D = 128
NW = 32
SEQ_PER_W = B // NW
L_PER_RND = 8
N_ROUNDS = L // L_PER_RND
ROWS_PER_XFER = 128
XFERS_PER_RND = 2
L_PER_XFER = 4
NSLOT = 3
ROWS_PER_W = N_ROUNDS * XFERS_PER_RND

_mesh = plsc.VectorSubcoreMesh(core_axis_name="c", subcore_axis_name="s")


@functools.partial(
    pl.kernel,
    mesh=_mesh,
    out_type=jax.ShapeDtypeStruct((B * L, D), jnp.float32),
    scratch_types=[
        pltpu.VMEM((NSLOT, XFERS_PER_RND, ROWS_PER_XFER), jnp.int32),
        pltpu.VMEM((NSLOT, XFERS_PER_RND, ROWS_PER_XFER), jnp.int32),
        pltpu.VMEM((NSLOT, L_PER_RND, D), jnp.float32),
        pltpu.VMEM((NSLOT, XFERS_PER_RND, ROWS_PER_XFER, D), jnp.float32),
        pltpu.SemaphoreType.DMA((NSLOT,)),
        pltpu.SemaphoreType.DMA((NSLOT,)),
        pltpu.SemaphoreType.DMA((NSLOT,)),
        pltpu.SemaphoreType.DMA((NSLOT,)),
    ],
)
def _emb_sc(xt_hbm, sym_hbm, pos_hbm, oidx_hbm, out_hbm,
            idx_v, oidx_v, pos_v, buf_v, sem_ip, sem_o, sem_g, sem_s):
    w = lax.axis_index("s") * 2 + lax.axis_index("c")

    def ip_copies(g, sl):
        row0 = w * ROWS_PER_W + g * XFERS_PER_RND
        return (
            pltpu.make_async_copy(xt_hbm.at[pl.ds(row0, XFERS_PER_RND)],
                                  idx_v.at[sl], sem_ip.at[sl]),
            pltpu.make_async_copy(pos_hbm.at[pl.ds(g * L_PER_RND, L_PER_RND)],
                                  pos_v.at[sl], sem_ip.at[sl]),
        )

    def o_copy(g, sl):
        row0 = w * ROWS_PER_W + g * XFERS_PER_RND
        return pltpu.make_async_copy(oidx_hbm.at[pl.ds(row0, XFERS_PER_RND)],
                                     oidx_v.at[sl], sem_o.at[sl])

    def g_copies(sl):
        return tuple(
            pltpu.make_async_copy(sym_hbm.at[idx_v.at[sl, t]],
                                  buf_v.at[sl, t], sem_g.at[sl])
            for t in range(XFERS_PER_RND))

    def s_copy(sl, t):
        return pltpu.make_async_copy(buf_v.at[sl, t],
                                     out_hbm.at[oidx_v.at[sl, t]],
                                     sem_s.at[sl])

    def s_copies(sl):
        return tuple(s_copy(sl, t) for t in range(XFERS_PER_RND))

    def start(cs):
        for c in cs:
            c.start()

    def wait(cs):
        for c in cs:
            c.wait()

    def adds_t(p, t):
        for li in range(L_PER_XFER):
            pvecs = [pos_v[p, t * L_PER_XFER + li, pl.ds(j * 16, 16)]
                     for j in range(8)]

            def sb_body(sb, carry, t=t, li=li, pvecs=pvecs):
                base = li * SEQ_PER_W + sb * 16
                for u in range(16):
                    for j in range(8):
                        plsc.addupdate(
                            buf_v.at[p, t, base + u, pl.ds(j * 16, 16)],
                            pvecs[j])
                return carry

            lax.fori_loop(0, 2, sb_body, 0)

    def round_step(g, phase):
        q = (phase + 1) % NSLOT
        wait(g_copies(phase))
        @pl.when(g + 2 < N_ROUNDS)
        def _():
            start(ip_copies(g + 2, (phase + 2) % NSLOT))
        @pl.when(g >= 2)
        def _():
            wait(s_copies(q))
        start((o_copy(g + 1, q),))
        wait(ip_copies(g + 1, q))
        start(g_copies(q))
        wait((o_copy(g, phase),))
        for t in range(XFERS_PER_RND):
            adds_t(phase, t)
            start((s_copy(phase, t),))

    start(ip_copies(0, 0))
    start((o_copy(0, 0),))
    wait(ip_copies(0, 0))
    start(g_copies(0))
    start(ip_copies(1, 1))

    def body(k, carry):
        for c in range(NSLOT):
            round_step(NSLOT * k + c, c)
        return carry

    lax.fori_loop(0, (N_ROUNDS - 1) // NSLOT, body, 0)

    g_last = N_ROUNDS - 1
    wait(g_copies(0))
    wait((o_copy(g_last, 0),))
    for t in range(XFERS_PER_RND):
        adds_t(0, t)
        start((s_copy(0, t),))
    wait(s_copies(1))
    wait(s_copies(2))
    wait(s_copies(0))


def _lmajor(a):
    return a.reshape(NW, SEQ_PER_W, L).swapaxes(1, 2).reshape(
        NW * ROWS_PER_W, ROWS_PER_XFER)


def kernel(x, sym_table, pos_table):
    xt = _lmajor(x.astype(jnp.int32))
    oidx = _lmajor(jnp.arange(B * L, dtype=jnp.int32))
    out = _emb_sc(xt, sym_table, pos_table, oidx)
    return out.reshape(B, L, D)
